# Initial kernel scaffold; baseline (speedup 1.0000x reference)
import jax
import jax.numpy as jnp
from jax.experimental import pallas as pl


def kernel(species, coordinates):
    raise NotImplementedError("write your pallas kernel here")


# speedup vs baseline: 3.4481x; 3.4481x over previous
import functools

import jax
import jax.numpy as jnp
import numpy as np
from jax.experimental import pallas as pl

_RCR = 5.2
_RCA = 3.5
_NUM_SPECIES = 4
_NUM_PAIRS = 10
_ETA_R = 16.0
_ETA_A = 8.0
_ZETA = 32
_SHF_R = np.array([0.9, 1.16875, 1.4375, 1.70625, 1.975, 2.24375, 2.5125,
                   2.78125, 3.05, 3.31875, 3.5875, 3.85625, 4.125, 4.39375,
                   4.6625, 4.93125], dtype=np.float32)
_SHF_A = np.array([0.9, 1.55, 2.2, 2.85], dtype=np.float32)
_SHF_Z = np.array([0.19634954, 0.58904862, 0.9817477, 1.3744468, 1.7671459,
                   2.1598449, 2.552544, 2.9452431], dtype=np.float32)
_A = 32
_CI = 8
_RADIAL_F = _NUM_SPECIES * 16
_ANGULAR_F = _NUM_PAIRS * 4 * 8


def _aev_body(species_ref, coords_ref, coords_c_ref, pidx_ref, out_ref):
    ci = pl.program_id(1)
    i0 = ci * _CI

    sp = species_ref[0, 0, :]
    pos = coords_ref[0, :, :]
    pos_c = coords_c_ref[0, 0, :, :]

    pj = pos[:, :, None]
    pk = pos[:, None, :]
    diff = pj - pk
    d2 = jnp.sum(diff * diff, axis=0)

    rows = jax.lax.broadcasted_iota(jnp.int32, (_A, _A), 0)
    cols = jax.lax.broadcasted_iota(jnp.int32, (_A, _A), 1)

    diff_c = pos_c[:, :, None] - pos[:, None, :]
    d2_c = jnp.sum(diff_c * diff_c, axis=0)
    crow = jax.lax.broadcasted_iota(jnp.int32, (_CI, _A), 0) + i0
    ccol = jax.lax.broadcasted_iota(jnp.int32, (_CI, _A), 1)
    diag_c = (crow == ccol)
    notdiag_c = (~diag_c).astype(jnp.float32)
    d_c = jnp.sqrt(d2_c + diag_c.astype(jnp.float32))
    inv_c = 1.0 / d_c

    pi = np.float32(np.pi)
    fc_r = jnp.where(d_c <= _RCR, 0.5 * jnp.cos(pi * d_c / _RCR) + 0.5, 0.0)
    fc_r = fc_r * notdiag_c
    shfr = (0.9 + 0.26875
            * jax.lax.broadcasted_iota(jnp.int32, (1, 1, 16), 2)
            .astype(jnp.float32))
    rad = 0.25 * jnp.exp(-_ETA_R * (d_c[:, :, None] - shfr) ** 2)
    rad = rad * fc_r[:, :, None]
    sidx = jax.lax.broadcasted_iota(jnp.int32, (_A, _NUM_SPECIES), 1)
    oh = (sp[:, None] == sidx).astype(jnp.float32)
    rad_st = jax.lax.dot_general(oh, rad, (((0,), (1,)), ((), ())),
                                 preferred_element_type=jnp.float32)
    radial = jnp.transpose(rad_st, (1, 0, 2)).reshape(_CI, _RADIAL_F)

    fc_a = jnp.where(d_c <= _RCA, 0.5 * jnp.cos(pi * d_c / _RCA) + 0.5, 0.0)
    fc_a = fc_a * notdiag_c

    dot = 0.5 * (d2_c[:, :, None] + d2_c[:, None, :] - d2[None, :, :])
    cth = 0.95 * dot * inv_c[:, :, None] * inv_c[:, None, :]
    sth = jnp.sqrt(jnp.maximum(1.0 - cth * cth, 0.0))
    dsum = 0.5 * (d_c[:, :, None] + d_c[:, None, :])
    jk = (rows < cols).astype(jnp.float32)
    w = 2.0 * fc_a[:, :, None] * fc_a[:, None, :] * jk[None, :, :]

    shfa = (0.9 + 0.65
            * jax.lax.broadcasted_iota(jnp.int32, (1, 4, 1, 1), 1)
            .astype(jnp.float32))
    f2 = jnp.exp(-_ETA_A * (dsum[:, None, :, :] - shfa) ** 2)
    shfz = ((pi / 16.0) + (pi / 8.0)
            * jax.lax.broadcasted_iota(jnp.int32, (1, 8, 1, 1), 1)
            .astype(jnp.float32))
    cz = jnp.cos(shfz)
    sz = jnp.sin(shfz)
    base = 0.5 + 0.5 * (cth[:, None, :, :] * cz + sth[:, None, :, :] * sz)
    f1 = base * base
    f1 = f1 * f1
    f1 = f1 * f1
    f1 = f1 * f1
    f1 = f1 * f1

    ang = (w[:, None, None, :, :] * f2[:, :, None, :, :]
           * f1[:, None, :, :, :])
    ang = ang.reshape(_CI, 32, _A * _A)

    pidx = jnp.broadcast_to(pidx_ref[0, :, :], (_NUM_PAIRS, _A * _A))
    pslot = jax.lax.broadcasted_iota(jnp.int32, (_NUM_PAIRS, _A * _A), 0)
    p_oht = (pidx == pslot).astype(jnp.float32)

    ang_p = jax.lax.dot_general(p_oht, ang, (((1,), (2,)), ((), ())),
                                preferred_element_type=jnp.float32)
    angular = jnp.transpose(ang_p, (1, 0, 2)).reshape(_CI, _ANGULAR_F)

    out_ref[0, :, :] = jnp.concatenate([radial, angular], axis=1)


@jax.jit
def _aev_pallas(species, coordinates):
    M, A = species.shape
    sp3 = species.astype(jnp.int32).reshape(M, 1, A)
    coords_t = jnp.transpose(coordinates, (0, 2, 1))
    coords_chunks = jnp.transpose(
        coords_t.reshape(M, 3, A // _CI, _CI), (0, 2, 1, 3))
    spi = species.astype(jnp.int32)
    mn = jnp.minimum(spi[:, :, None], spi[:, None, :])
    mx = jnp.maximum(spi[:, :, None], spi[:, None, :])
    pidx_flat = ((mn * (7 - mn)) // 2 + mx).reshape(M, 1, A * A)
    out = pl.pallas_call(
        _aev_body,
        grid=(M, A // _CI),
        in_specs=[
            pl.BlockSpec((1, 1, A), lambda m, c: (m, 0, 0)),
            pl.BlockSpec((1, 3, A), lambda m, c: (m, 0, 0)),
            pl.BlockSpec((1, 1, 3, _CI), lambda m, c: (m, c, 0, 0)),
            pl.BlockSpec((1, 1, A * A), lambda m, c: (m, 0, 0)),
        ],
        out_specs=pl.BlockSpec((1, _CI, _RADIAL_F + _ANGULAR_F),
                               lambda m, c: (m, c, 0)),
        out_shape=jax.ShapeDtypeStruct((M, A, _RADIAL_F + _ANGULAR_F),
                                       jnp.float32),
    )(sp3, coords_t, coords_chunks, pidx_flat)
    return out


def kernel(species, coordinates):
    aev = _aev_pallas(species, coordinates)
    return (species, aev)


# speedup vs baseline: 6.3190x; 1.8326x over previous
import functools

import jax
import jax.numpy as jnp
import numpy as np
from jax.experimental import pallas as pl

_RCR = 5.2
_RCA = 3.5
_NUM_SPECIES = 4
_NUM_PAIRS = 10
_ETA_R = 16.0
_ETA_A = 8.0
_A = 32
_AA = _A * _A
_CI = 8
_RADIAL_F = _NUM_SPECIES * 16
_ANGULAR_F = _NUM_PAIRS * 4 * 8


def _aev_body(species_ref, coords_ref, coords_c_ref, posj_ref, posk_ref,
              pidx_ref, out_ref):
    ci = pl.program_id(1)
    i0 = ci * _CI
    pi = np.float32(np.pi)

    sp = species_ref[0, 0, :]
    pos = coords_ref[0, :, :]
    pos_c = coords_c_ref[0, 0, :, :]

    diff_c = pos_c[:, :, None] - pos[:, None, :]
    d2_c = jnp.sum(diff_c * diff_c, axis=0)
    crow = jax.lax.broadcasted_iota(jnp.int32, (_CI, _A), 0) + i0
    ccol = jax.lax.broadcasted_iota(jnp.int32, (_CI, _A), 1)
    diag_c = (crow == ccol)
    notdiag_c = (~diag_c).astype(jnp.float32)
    d_c = jnp.sqrt(d2_c + diag_c.astype(jnp.float32))

    fc_r = jnp.where(d_c <= _RCR, 0.5 * jnp.cos(pi * d_c / _RCR) + 0.5, 0.0)
    fc_r = fc_r * notdiag_c
    shfr = (0.9 + 0.26875
            * jax.lax.broadcasted_iota(jnp.int32, (1, 1, 16), 2)
            .astype(jnp.float32))
    rad = 0.25 * jnp.exp(-_ETA_R * (d_c[:, :, None] - shfr) ** 2)
    rad = rad * fc_r[:, :, None]
    sidx = jax.lax.broadcasted_iota(jnp.int32, (_A, _NUM_SPECIES), 1)
    oh = (sp[:, None] == sidx).astype(jnp.float32)
    rad_st = jax.lax.dot_general(oh, rad, (((0,), (1,)), ((), ())),
                                 preferred_element_type=jnp.float32)
    radial = jnp.transpose(rad_st, (1, 0, 2)).reshape(_CI, _RADIAL_F)

    posj = posj_ref[0, :, :]
    posk = posk_ref[0, :, :]

    dj = pos_c[:, :, None] - posj[:, None, :]
    d2_ij = jnp.sum(dj * dj, axis=0)
    dk = pos_c[:, :, None] - posk[:, None, :]
    d2_ik = jnp.sum(dk * dk, axis=0)
    ejk = posj - posk
    e2 = ejk * ejk
    d2_jk = e2[0:1, :] + e2[1:2, :] + e2[2:3, :]

    qi = jax.lax.broadcasted_iota(jnp.int32, (1, _AA), 1)
    jq = qi // _A
    kq = qi - jq * _A
    irow = jax.lax.broadcasted_iota(jnp.int32, (_CI, 1), 0) + i0

    diag_ij = (jq == irow).astype(jnp.float32)
    diag_ik = (kq == irow).astype(jnp.float32)
    d_ij = jnp.sqrt(d2_ij + diag_ij)
    d_ik = jnp.sqrt(d2_ik + diag_ik)
    inv_ij = 1.0 / d_ij
    inv_ik = 1.0 / d_ik

    fc_ij = jnp.where(d_ij <= _RCA, 0.5 * jnp.cos(pi * d_ij / _RCA) + 0.5,
                      0.0) * (1.0 - diag_ij)
    fc_ik = jnp.where(d_ik <= _RCA, 0.5 * jnp.cos(pi * d_ik / _RCA) + 0.5,
                      0.0) * (1.0 - diag_ik)
    tri = (jq < kq).astype(jnp.float32)
    w = 2.0 * fc_ij * fc_ik * tri

    dotv = 0.5 * (d2_ij + d2_ik - d2_jk)
    cth = 0.95 * dotv * inv_ij * inv_ik
    sth = jnp.sqrt(jnp.maximum(1.0 - cth * cth, 0.0))
    dsum = 0.5 * (d_ij + d_ik)

    shfa = (0.9 + 0.65
            * jax.lax.broadcasted_iota(jnp.int32, (1, 4, 1), 1)
            .astype(jnp.float32))
    f2 = jnp.exp(-_ETA_A * (dsum[:, None, :] - shfa) ** 2)
    shfz = ((pi / 16.0) + (pi / 8.0)
            * jax.lax.broadcasted_iota(jnp.int32, (1, 8, 1), 1)
            .astype(jnp.float32))
    cz = jnp.cos(shfz)
    sz = jnp.sin(shfz)
    base = 0.5 + 0.5 * (cth[:, None, :] * cz + sth[:, None, :] * sz)
    f1 = base * base
    f1 = f1 * f1
    f1 = f1 * f1
    f1 = f1 * f1
    f1 = f1 * f1

    ang = (w[:, None, None, :] * f2[:, :, None, :]
           * f1[:, None, :, :])
    ang = ang.reshape(_CI, 32, _AA)

    pidx = jnp.broadcast_to(pidx_ref[0, :, :], (_NUM_PAIRS, _AA))
    pslot = jax.lax.broadcasted_iota(jnp.int32, (_NUM_PAIRS, _AA), 0)
    p_oht = (pidx == pslot).astype(jnp.float32)

    ang_p = jax.lax.dot_general(p_oht, ang, (((1,), (2,)), ((), ())),
                                preferred_element_type=jnp.float32)
    angular = jnp.transpose(ang_p, (1, 0, 2)).reshape(_CI, _ANGULAR_F)

    out_ref[0, :, :] = jnp.concatenate([radial, angular], axis=1)


@jax.jit
def _aev_pallas(species, coordinates):
    M, A = species.shape
    sp3 = species.astype(jnp.int32).reshape(M, 1, A)
    coords_t = jnp.transpose(coordinates, (0, 2, 1))
    coords_chunks = jnp.transpose(
        coords_t.reshape(M, 3, A // _CI, _CI), (0, 2, 1, 3))
    posj = jnp.repeat(coords_t, A, axis=2)
    posk = jnp.tile(coords_t, (1, 1, A))
    spi = species.astype(jnp.int32)
    mn = jnp.minimum(spi[:, :, None], spi[:, None, :])
    mx = jnp.maximum(spi[:, :, None], spi[:, None, :])
    pidx_flat = ((mn * (7 - mn)) // 2 + mx).reshape(M, 1, A * A)
    out = pl.pallas_call(
        _aev_body,
        grid=(M, A // _CI),
        in_specs=[
            pl.BlockSpec((1, 1, A), lambda m, c: (m, 0, 0)),
            pl.BlockSpec((1, 3, A), lambda m, c: (m, 0, 0)),
            pl.BlockSpec((1, 1, 3, _CI), lambda m, c: (m, c, 0, 0)),
            pl.BlockSpec((1, 3, A * A), lambda m, c: (m, 0, 0)),
            pl.BlockSpec((1, 3, A * A), lambda m, c: (m, 0, 0)),
            pl.BlockSpec((1, 1, A * A), lambda m, c: (m, 0, 0)),
        ],
        out_specs=pl.BlockSpec((1, _CI, _RADIAL_F + _ANGULAR_F),
                               lambda m, c: (m, c, 0)),
        out_shape=jax.ShapeDtypeStruct((M, A, _RADIAL_F + _ANGULAR_F),
                                       jnp.float32),
    )(sp3, coords_t, coords_chunks, posj, posk, pidx_flat)
    return out


def kernel(species, coordinates):
    aev = _aev_pallas(species, coordinates)
    return (species, aev)


# speedup vs baseline: 7.3151x; 1.1576x over previous
import functools

import jax
import jax.numpy as jnp
import numpy as np
from jax.experimental import pallas as pl

_RCR = 5.2
_RCA = 3.5
_NUM_SPECIES = 4
_NUM_PAIRS = 10
_ETA_R = 16.0
_ETA_A = 8.0
_A = 32
_NQ = 512
_NPAIR = _A * (_A - 1) // 2
_CI = 8
_RADIAL_F = _NUM_SPECIES * 16
_ANGULAR_F = _NUM_PAIRS * 4 * 8

_JQ, _KQ = np.triu_indices(_A, k=1)


def _aev_body(species_ref, coords_ref, coords_c_ref, posj_ref, posk_ref,
              pidx_ref, jq_ref, kq_ref, out_ref):
    ci = pl.program_id(1)
    i0 = ci * _CI
    pi = np.float32(np.pi)

    sp = species_ref[0, 0, :]
    pos = coords_ref[0, :, :]
    pos_c = coords_c_ref[0, 0, :, :]

    diff_c = pos_c[:, :, None] - pos[:, None, :]
    d2_c = jnp.sum(diff_c * diff_c, axis=0)
    crow = jax.lax.broadcasted_iota(jnp.int32, (_CI, _A), 0) + i0
    ccol = jax.lax.broadcasted_iota(jnp.int32, (_CI, _A), 1)
    diag_c = (crow == ccol)
    notdiag_c = (~diag_c).astype(jnp.float32)
    d_c = jnp.sqrt(d2_c + diag_c.astype(jnp.float32))

    fc_r = jnp.where(d_c <= _RCR, 0.5 * jnp.cos(pi * d_c / _RCR) + 0.5, 0.0)
    fc_r = fc_r * notdiag_c
    shfr = (0.9 + 0.26875
            * jax.lax.broadcasted_iota(jnp.int32, (1, 1, 16), 2)
            .astype(jnp.float32))
    rad = 0.25 * jnp.exp(-_ETA_R * (d_c[:, :, None] - shfr) ** 2)
    rad = rad * fc_r[:, :, None]
    sidx = jax.lax.broadcasted_iota(jnp.int32, (_A, _NUM_SPECIES), 1)
    oh = (sp[:, None] == sidx).astype(jnp.float32)
    rad_st = jax.lax.dot_general(oh, rad, (((0,), (1,)), ((), ())),
                                 preferred_element_type=jnp.float32)
    radial = jnp.transpose(rad_st, (1, 0, 2)).reshape(_CI, _RADIAL_F)

    posj = posj_ref[0, :, :]
    posk = posk_ref[0, :, :]
    jq = jq_ref[0, :, :]
    kq = kq_ref[0, :, :]

    dj = pos_c[:, :, None] - posj[:, None, :]
    d2_ij = jnp.sum(dj * dj, axis=0)
    dk = pos_c[:, :, None] - posk[:, None, :]
    d2_ik = jnp.sum(dk * dk, axis=0)
    ejk = posj - posk
    e2 = ejk * ejk
    d2_jk = e2[0:1, :] + e2[1:2, :] + e2[2:3, :]

    irow = jax.lax.broadcasted_iota(jnp.int32, (_CI, 1), 0) + i0
    diag_ij = (jq == irow).astype(jnp.float32)
    diag_ik = (kq == irow).astype(jnp.float32)

    s2_ij = d2_ij + diag_ij
    s2_ik = d2_ik + diag_ik
    inv_ij = jax.lax.rsqrt(s2_ij)
    inv_ik = jax.lax.rsqrt(s2_ik)
    d_ij = s2_ij * inv_ij
    d_ik = s2_ik * inv_ik

    fc_ij = jnp.where(d_ij <= _RCA, 0.5 * jnp.cos(pi * d_ij / _RCA) + 0.5,
                      0.0) * (1.0 - diag_ij)
    fc_ik = jnp.where(d_ik <= _RCA, 0.5 * jnp.cos(pi * d_ik / _RCA) + 0.5,
                      0.0) * (1.0 - diag_ik)
    w = 2.0 * fc_ij * fc_ik

    dotv = 0.5 * (d2_ij + d2_ik - d2_jk)
    cth = 0.95 * dotv * inv_ij * inv_ik
    sth = jnp.sqrt(jnp.maximum(1.0 - cth * cth, 0.0))
    dsum = jnp.minimum(0.5 * (d_ij + d_ik), 4.0)

    f2_0 = jnp.exp(-_ETA_A * (dsum - 0.9) ** 2)
    r = jnp.exp(10.4 * dsum)
    f2_1 = f2_0 * (r * np.float32(np.exp(-10.4 * 0.9 - 3.38)))
    f2_2 = f2_1 * (r * np.float32(np.exp(-10.4 * 1.55 - 3.38)))
    f2_3 = f2_2 * (r * np.float32(np.exp(-10.4 * 2.2 - 3.38)))
    wf2 = jnp.stack([w * f2_0, w * f2_1, w * f2_2, w * f2_3],
                    axis=1)

    shfz = ((pi / 16.0) + (pi / 8.0)
            * jax.lax.broadcasted_iota(jnp.int32, (1, 8, 1), 1)
            .astype(jnp.float32))
    czh = 0.5 * jnp.cos(shfz)
    szh = 0.5 * jnp.sin(shfz)
    base = 0.5 + cth[:, None, :] * czh + sth[:, None, :] * szh
    f1 = base * base
    f1 = f1 * f1
    f1 = f1 * f1
    f1 = f1 * f1
    f1 = f1 * f1

    ang = wf2[:, :, None, :] * f1[:, None, :, :]
    ang = ang.reshape(_CI, 32, _NQ)

    pidx = jnp.broadcast_to(pidx_ref[0, :, :], (_NUM_PAIRS, _NQ))
    pslot = jax.lax.broadcasted_iota(jnp.int32, (_NUM_PAIRS, _NQ), 0)
    p_oht = (pidx == pslot).astype(jnp.float32)

    ang_p = jax.lax.dot_general(p_oht, ang, (((1,), (2,)), ((), ())),
                                preferred_element_type=jnp.float32)
    angular = jnp.transpose(ang_p, (1, 0, 2)).reshape(_CI, _ANGULAR_F)

    out_ref[0, :, :] = jnp.concatenate([radial, angular], axis=1)


@jax.jit
def _aev_pallas(species, coordinates):
    M, A = species.shape
    sp3 = species.astype(jnp.int32).reshape(M, 1, A)
    coords_t = jnp.transpose(coordinates, (0, 2, 1))
    coords_chunks = jnp.transpose(
        coords_t.reshape(M, 3, A // _CI, _CI), (0, 2, 1, 3))

    jq = jnp.asarray(_JQ, dtype=jnp.int32)
    kq = jnp.asarray(_KQ, dtype=jnp.int32)
    npad = _NQ - _NPAIR
    posj = jnp.concatenate(
        [jnp.take(coords_t, jq, axis=2),
         jnp.full((M, 3, npad), 1.0e4, jnp.float32)], axis=2)
    posk = jnp.concatenate(
        [jnp.take(coords_t, kq, axis=2),
         jnp.full((M, 3, npad), 2.0e4, jnp.float32)], axis=2)

    spi = species.astype(jnp.int32)
    spj = jnp.take(spi, jq, axis=1)
    spk = jnp.take(spi, kq, axis=1)
    mn = jnp.minimum(spj, spk)
    mx = jnp.maximum(spj, spk)
    pidx = (mn * (7 - mn)) // 2 + mx
    pidx = jnp.pad(pidx, ((0, 0), (0, npad))).reshape(M, 1, _NQ)

    jq_arr = jnp.pad(jq, (0, npad)).reshape(1, 1, _NQ)
    kq_arr = jnp.pad(kq, (0, npad)).reshape(1, 1, _NQ)

    out = pl.pallas_call(
        _aev_body,
        grid=(M, A // _CI),
        in_specs=[
            pl.BlockSpec((1, 1, A), lambda m, c: (m, 0, 0)),
            pl.BlockSpec((1, 3, A), lambda m, c: (m, 0, 0)),
            pl.BlockSpec((1, 1, 3, _CI), lambda m, c: (m, c, 0, 0)),
            pl.BlockSpec((1, 3, _NQ), lambda m, c: (m, 0, 0)),
            pl.BlockSpec((1, 3, _NQ), lambda m, c: (m, 0, 0)),
            pl.BlockSpec((1, 1, _NQ), lambda m, c: (m, 0, 0)),
            pl.BlockSpec((1, 1, _NQ), lambda m, c: (0, 0, 0)),
            pl.BlockSpec((1, 1, _NQ), lambda m, c: (0, 0, 0)),
        ],
        out_specs=pl.BlockSpec((1, _CI, _RADIAL_F + _ANGULAR_F),
                               lambda m, c: (m, c, 0)),
        out_shape=jax.ShapeDtypeStruct((M, A, _RADIAL_F + _ANGULAR_F),
                                       jnp.float32),
    )(sp3, coords_t, coords_chunks, posj, posk, pidx, jq_arr, kq_arr)
    return out


def kernel(species, coordinates):
    aev = _aev_pallas(species, coordinates)
    return (species, aev)


# speedup vs baseline: 9.6580x; 1.3203x over previous
import functools

import jax
import jax.numpy as jnp
import numpy as np
from jax.experimental import pallas as pl

_RCR = 5.2
_RCA = 3.5
_NUM_SPECIES = 4
_NUM_PAIRS = 10
_ETA_R = 16.0
_ETA_A = 8.0
_A = 32
_NQ = 512
_NPAIR = _A * (_A - 1) // 2
_CI = 16
_RADIAL_F = _NUM_SPECIES * 16
_ANGULAR_F = _NUM_PAIRS * 4 * 8

_JQ, _KQ = np.triu_indices(_A, k=1)


def _aev_body(species_ref, coords_ref, coords_c_ref, posj_ref, posk_ref,
              pidx_ref, dgj_ref, dgk_ref, out_ref):
    ci = pl.program_id(1)
    i0 = ci * _CI
    pi = np.float32(np.pi)

    sp = species_ref[0, 0, :]
    pos = coords_ref[0, :, :]
    pos_c = coords_c_ref[0, 0, :, :]

    diff_c = pos_c[:, :, None] - pos[:, None, :]
    d2_c = jnp.sum(diff_c * diff_c, axis=0)
    crow = jax.lax.broadcasted_iota(jnp.int32, (_CI, _A), 0) + i0
    ccol = jax.lax.broadcasted_iota(jnp.int32, (_CI, _A), 1)
    diag_c = (crow == ccol)
    notdiag_c = (~diag_c).astype(jnp.float32)
    d_c = jnp.sqrt(d2_c + diag_c.astype(jnp.float32))

    fc_r = jnp.where(d_c <= _RCR, 0.5 * jnp.cos(pi * d_c / _RCR) + 0.5, 0.0)
    fc_r = fc_r * notdiag_c
    shfr = (0.9 + 0.26875
            * jax.lax.broadcasted_iota(jnp.int32, (1, 1, 16), 2)
            .astype(jnp.float32))
    rad = 0.25 * jnp.exp(-_ETA_R * (d_c[:, :, None] - shfr) ** 2)
    rad = rad * fc_r[:, :, None]
    sidx = jax.lax.broadcasted_iota(jnp.int32, (_A, _NUM_SPECIES), 1)
    oh = (sp[:, None] == sidx).astype(jnp.float32)
    rad_st = jax.lax.dot_general(oh, rad, (((0,), (1,)), ((), ())),
                                 preferred_element_type=jnp.float32)
    radial = jnp.transpose(rad_st, (1, 0, 2)).reshape(_CI, _RADIAL_F)

    posj = posj_ref[0, :, :]
    posk = posk_ref[0, :, :]
    diag_ij = dgj_ref[0, :, :]
    diag_ik = dgk_ref[0, :, :]

    dj = pos_c[:, :, None] - posj[:, None, :]
    d2_ij = jnp.sum(dj * dj, axis=0)
    dk = pos_c[:, :, None] - posk[:, None, :]
    d2_ik = jnp.sum(dk * dk, axis=0)
    ejk = posj - posk
    e2 = ejk * ejk
    d2_jk = e2[0:1, :] + e2[1:2, :] + e2[2:3, :]

    s2_ij = d2_ij + diag_ij
    s2_ik = d2_ik + diag_ik
    inv_ij = jax.lax.rsqrt(s2_ij)
    inv_ik = jax.lax.rsqrt(s2_ik)
    d_ij = s2_ij * inv_ij
    d_ik = s2_ik * inv_ik

    fc_ij = jnp.where(d_ij <= _RCA, 0.5 * jnp.cos(pi * d_ij / _RCA) + 0.5,
                      0.0) * (1.0 - diag_ij)
    fc_ik = jnp.where(d_ik <= _RCA, 0.5 * jnp.cos(pi * d_ik / _RCA) + 0.5,
                      0.0) * (1.0 - diag_ik)
    w = 2.0 * fc_ij * fc_ik

    dotv = 0.5 * (d2_ij + d2_ik - d2_jk)
    cth = 0.95 * dotv * inv_ij * inv_ik
    sth = jnp.sqrt(jnp.maximum(1.0 - cth * cth, 0.0))
    dsum = jnp.minimum(0.5 * (d_ij + d_ik), 4.0)

    f2_0 = jnp.exp(-_ETA_A * (dsum - 0.9) ** 2)
    r = jnp.exp(10.4 * dsum)
    f2_1 = f2_0 * (r * np.float32(np.exp(-10.4 * 0.9 - 3.38)))
    f2_2 = f2_1 * (r * np.float32(np.exp(-10.4 * 1.55 - 3.38)))
    f2_3 = f2_2 * (r * np.float32(np.exp(-10.4 * 2.2 - 3.38)))
    wf2 = jnp.stack([w * f2_0, w * f2_1, w * f2_2, w * f2_3],
                    axis=1)

    shfz = ((pi / 16.0) + (pi / 8.0)
            * jax.lax.broadcasted_iota(jnp.int32, (1, 8, 1), 1)
            .astype(jnp.float32))
    czh = 0.5 * jnp.cos(shfz)
    szh = 0.5 * jnp.sin(shfz)
    base = 0.5 + cth[:, None, :] * czh + sth[:, None, :] * szh
    f1 = base * base
    f1 = f1 * f1
    f1 = f1 * f1
    f1 = f1 * f1
    f1 = f1 * f1

    ang = wf2[:, :, None, :] * f1[:, None, :, :]
    ang = ang.reshape(_CI, 32, _NQ)

    pidx = jnp.broadcast_to(pidx_ref[0, :, :], (_NUM_PAIRS, _NQ))
    pslot = jax.lax.broadcasted_iota(jnp.int32, (_NUM_PAIRS, _NQ), 0)
    p_oht = (pidx == pslot).astype(jnp.float32)

    ang_p = jax.lax.dot_general(p_oht, ang, (((1,), (2,)), ((), ())),
                                preferred_element_type=jnp.float32)
    angular = jnp.transpose(ang_p, (1, 0, 2)).reshape(_CI, _ANGULAR_F)

    out_ref[0, :, :] = jnp.concatenate([radial, angular], axis=1)


@jax.jit
def _aev_pallas(species, coordinates):
    M, A = species.shape
    sp3 = species.astype(jnp.int32).reshape(M, 1, A)
    coords_t = jnp.transpose(coordinates, (0, 2, 1))
    coords_chunks = jnp.transpose(
        coords_t.reshape(M, 3, A // _CI, _CI), (0, 2, 1, 3))

    jq = jnp.asarray(_JQ, dtype=jnp.int32)
    kq = jnp.asarray(_KQ, dtype=jnp.int32)
    npad = _NQ - _NPAIR
    posj = jnp.concatenate(
        [jnp.take(coords_t, jq, axis=2),
         jnp.full((M, 3, npad), 1.0e4, jnp.float32)], axis=2)
    posk = jnp.concatenate(
        [jnp.take(coords_t, kq, axis=2),
         jnp.full((M, 3, npad), 2.0e4, jnp.float32)], axis=2)

    spi = species.astype(jnp.int32)
    spj = jnp.take(spi, jq, axis=1)
    spk = jnp.take(spi, kq, axis=1)
    mn = jnp.minimum(spj, spk)
    mx = jnp.maximum(spj, spk)
    pidx = (mn * (7 - mn)) // 2 + mx
    pidx = jnp.pad(pidx, ((0, 0), (0, npad))).reshape(M, 1, _NQ)

    jq_pad = np.pad(_JQ, (0, npad), constant_values=-1)
    kq_pad = np.pad(_KQ, (0, npad), constant_values=-1)
    irows = np.arange(A)[:, None]
    dgj = jnp.asarray((jq_pad[None, :] == irows).astype(np.float32)
                      ).reshape(1, A, _NQ)
    dgk = jnp.asarray((kq_pad[None, :] == irows).astype(np.float32)
                      ).reshape(1, A, _NQ)

    out = pl.pallas_call(
        _aev_body,
        grid=(M, A // _CI),
        in_specs=[
            pl.BlockSpec((1, 1, A), lambda m, c: (m, 0, 0)),
            pl.BlockSpec((1, 3, A), lambda m, c: (m, 0, 0)),
            pl.BlockSpec((1, 1, 3, _CI), lambda m, c: (m, c, 0, 0)),
            pl.BlockSpec((1, 3, _NQ), lambda m, c: (m, 0, 0)),
            pl.BlockSpec((1, 3, _NQ), lambda m, c: (m, 0, 0)),
            pl.BlockSpec((1, 1, _NQ), lambda m, c: (m, 0, 0)),
            pl.BlockSpec((1, _CI, _NQ), lambda m, c: (0, c, 0)),
            pl.BlockSpec((1, _CI, _NQ), lambda m, c: (0, c, 0)),
        ],
        out_specs=pl.BlockSpec((1, _CI, _RADIAL_F + _ANGULAR_F),
                               lambda m, c: (m, c, 0)),
        out_shape=jax.ShapeDtypeStruct((M, A, _RADIAL_F + _ANGULAR_F),
                                       jnp.float32),
    )(sp3, coords_t, coords_chunks, posj, posk, pidx, dgj, dgk)
    return out


def kernel(species, coordinates):
    aev = _aev_pallas(species, coordinates)
    return (species, aev)


# speedup vs baseline: 11.0248x; 1.1415x over previous
import functools

import jax
import jax.numpy as jnp
import numpy as np
from jax.experimental import pallas as pl

_RCR = 5.2
_RCA = 3.5
_NUM_SPECIES = 4
_NUM_PAIRS = 10
_ETA_R = 16.0
_ETA_A = 8.0
_A = 32
_NQ = 512
_NPAIR = _A * (_A - 1) // 2
_CI = 32
_RADIAL_F = _NUM_SPECIES * 16
_ANGULAR_F = _NUM_PAIRS * 4 * 8

_JQ, _KQ = np.triu_indices(_A, k=1)


def _aev_body(species_ref, coords_ref, coords_c_ref, posj_ref, posk_ref,
              pidx_ref, dgj_ref, dgk_ref, out_ref):
    ci = pl.program_id(1)
    i0 = ci * _CI
    pi = np.float32(np.pi)

    sp = species_ref[0, 0, :]
    pos = coords_ref[0, :, :]
    pos_c = coords_c_ref[0, 0, :, :]

    diff_c = pos_c[:, :, None] - pos[:, None, :]
    d2_c = jnp.sum(diff_c * diff_c, axis=0)
    crow = jax.lax.broadcasted_iota(jnp.int32, (_CI, _A), 0) + i0
    ccol = jax.lax.broadcasted_iota(jnp.int32, (_CI, _A), 1)
    diag_c = (crow == ccol)
    notdiag_c = (~diag_c).astype(jnp.float32)
    d_c = jnp.sqrt(d2_c + diag_c.astype(jnp.float32))

    fc_r = jnp.where(d_c <= _RCR, 0.5 * jnp.cos(pi * d_c / _RCR) + 0.5, 0.0)
    fc_r = fc_r * notdiag_c
    shfr = (0.9 + 0.26875
            * jax.lax.broadcasted_iota(jnp.int32, (1, 1, 16), 2)
            .astype(jnp.float32))
    rad = 0.25 * jnp.exp(-_ETA_R * (d_c[:, :, None] - shfr) ** 2)
    rad = rad * fc_r[:, :, None]
    sidx = jax.lax.broadcasted_iota(jnp.int32, (_A, _NUM_SPECIES), 1)
    oh = (sp[:, None] == sidx).astype(jnp.float32)
    rad_st = jax.lax.dot_general(oh, rad, (((0,), (1,)), ((), ())),
                                 preferred_element_type=jnp.float32)
    radial = jnp.transpose(rad_st, (1, 0, 2)).reshape(_CI, _RADIAL_F)

    posj = posj_ref[0, :, :]
    posk = posk_ref[0, :, :]
    diag_ij = dgj_ref[0, :, :]
    diag_ik = dgk_ref[0, :, :]

    dj = pos_c[:, :, None] - posj[:, None, :]
    d2_ij = jnp.sum(dj * dj, axis=0)
    dk = pos_c[:, :, None] - posk[:, None, :]
    d2_ik = jnp.sum(dk * dk, axis=0)
    ejk = posj - posk
    e2 = ejk * ejk
    d2_jk = e2[0:1, :] + e2[1:2, :] + e2[2:3, :]

    s2_ij = d2_ij + diag_ij
    s2_ik = d2_ik + diag_ik
    inv_ij = jax.lax.rsqrt(s2_ij)
    inv_ik = jax.lax.rsqrt(s2_ik)
    d_ij = s2_ij * inv_ij
    d_ik = s2_ik * inv_ik

    fc_ij = jnp.where(d_ij <= _RCA, 0.5 * jnp.cos(pi * d_ij / _RCA) + 0.5,
                      0.0) * (1.0 - diag_ij)
    fc_ik = jnp.where(d_ik <= _RCA, 0.5 * jnp.cos(pi * d_ik / _RCA) + 0.5,
                      0.0) * (1.0 - diag_ik)
    w = 2.0 * fc_ij * fc_ik

    dotv = 0.5 * (d2_ij + d2_ik - d2_jk)
    cth = 0.95 * dotv * inv_ij * inv_ik
    sth = jnp.sqrt(jnp.maximum(1.0 - cth * cth, 0.0))
    dsum = jnp.minimum(0.5 * (d_ij + d_ik), 4.0)

    f2_0 = jnp.exp(-_ETA_A * (dsum - 0.9) ** 2)
    r = jnp.exp(10.4 * dsum)
    f2_1 = f2_0 * (r * np.float32(np.exp(-10.4 * 0.9 - 3.38)))
    f2_2 = f2_1 * (r * np.float32(np.exp(-10.4 * 1.55 - 3.38)))
    f2_3 = f2_2 * (r * np.float32(np.exp(-10.4 * 2.2 - 3.38)))
    wf2 = jnp.stack([w * f2_0, w * f2_1, w * f2_2, w * f2_3],
                    axis=1)

    shfz = ((pi / 16.0) + (pi / 8.0)
            * jax.lax.broadcasted_iota(jnp.int32, (1, 8, 1), 1)
            .astype(jnp.float32))
    czh = 0.5 * jnp.cos(shfz)
    szh = 0.5 * jnp.sin(shfz)
    base = 0.5 + cth[:, None, :] * czh + sth[:, None, :] * szh
    f1 = base * base
    f1 = f1 * f1
    f1 = f1 * f1
    f1 = f1 * f1
    f1 = f1 * f1

    ang = wf2[:, :, None, :] * f1[:, None, :, :]
    ang = ang.reshape(_CI, 32, _NQ)

    pidx = jnp.broadcast_to(pidx_ref[0, :, :], (_NUM_PAIRS, _NQ))
    pslot = jax.lax.broadcasted_iota(jnp.int32, (_NUM_PAIRS, _NQ), 0)
    p_oht = (pidx == pslot).astype(jnp.float32)

    ang_p = jax.lax.dot_general(p_oht, ang, (((1,), (2,)), ((), ())),
                                preferred_element_type=jnp.float32)
    angular = jnp.transpose(ang_p, (1, 0, 2)).reshape(_CI, _ANGULAR_F)

    out_ref[0, :, :] = jnp.concatenate([radial, angular], axis=1)


@jax.jit
def _aev_pallas(species, coordinates):
    M, A = species.shape
    sp3 = species.astype(jnp.int32).reshape(M, 1, A)
    coords_t = jnp.transpose(coordinates, (0, 2, 1))
    coords_chunks = jnp.transpose(
        coords_t.reshape(M, 3, A // _CI, _CI), (0, 2, 1, 3))

    jq = jnp.asarray(_JQ, dtype=jnp.int32)
    kq = jnp.asarray(_KQ, dtype=jnp.int32)
    npad = _NQ - _NPAIR
    posj = jnp.concatenate(
        [jnp.take(coords_t, jq, axis=2),
         jnp.full((M, 3, npad), 1.0e4, jnp.float32)], axis=2)
    posk = jnp.concatenate(
        [jnp.take(coords_t, kq, axis=2),
         jnp.full((M, 3, npad), 2.0e4, jnp.float32)], axis=2)

    spi = species.astype(jnp.int32)
    spj = jnp.take(spi, jq, axis=1)
    spk = jnp.take(spi, kq, axis=1)
    mn = jnp.minimum(spj, spk)
    mx = jnp.maximum(spj, spk)
    pidx = (mn * (7 - mn)) // 2 + mx
    pidx = jnp.pad(pidx, ((0, 0), (0, npad))).reshape(M, 1, _NQ)

    jq_pad = np.pad(_JQ, (0, npad), constant_values=-1)
    kq_pad = np.pad(_KQ, (0, npad), constant_values=-1)
    irows = np.arange(A)[:, None]
    dgj = jnp.asarray((jq_pad[None, :] == irows).astype(np.float32)
                      ).reshape(1, A, _NQ)
    dgk = jnp.asarray((kq_pad[None, :] == irows).astype(np.float32)
                      ).reshape(1, A, _NQ)

    out = pl.pallas_call(
        _aev_body,
        grid=(M, A // _CI),
        in_specs=[
            pl.BlockSpec((1, 1, A), lambda m, c: (m, 0, 0)),
            pl.BlockSpec((1, 3, A), lambda m, c: (m, 0, 0)),
            pl.BlockSpec((1, 1, 3, _CI), lambda m, c: (m, c, 0, 0)),
            pl.BlockSpec((1, 3, _NQ), lambda m, c: (m, 0, 0)),
            pl.BlockSpec((1, 3, _NQ), lambda m, c: (m, 0, 0)),
            pl.BlockSpec((1, 1, _NQ), lambda m, c: (m, 0, 0)),
            pl.BlockSpec((1, _CI, _NQ), lambda m, c: (0, c, 0)),
            pl.BlockSpec((1, _CI, _NQ), lambda m, c: (0, c, 0)),
        ],
        out_specs=pl.BlockSpec((1, _CI, _RADIAL_F + _ANGULAR_F),
                               lambda m, c: (m, c, 0)),
        out_shape=jax.ShapeDtypeStruct((M, A, _RADIAL_F + _ANGULAR_F),
                                       jnp.float32),
    )(sp3, coords_t, coords_chunks, posj, posk, pidx, dgj, dgk)
    return out


def kernel(species, coordinates):
    aev = _aev_pallas(species, coordinates)
    return (species, aev)


# speedup vs baseline: 12.5979x; 1.1427x over previous
import functools

import jax
import jax.numpy as jnp
import numpy as np
from jax.experimental import pallas as pl

_RCR = 5.2
_RCA = 3.5
_NUM_SPECIES = 4
_NUM_PAIRS = 10
_ETA_R = 16.0
_ETA_A = 8.0
_A = 32
_NQ = 512
_NPAIR = _A * (_A - 1) // 2
_CI = 32
_RADIAL_F = _NUM_SPECIES * 16
_ANGULAR_F = _NUM_PAIRS * 4 * 8

_JQ, _KQ = np.triu_indices(_A, k=1)

_FC_COEF = (9.9999998695e-01, -2.4674003665e+00, 2.0293461123e+00,
            -6.6757576357e-01, 1.1751096555e-01, -1.2677815461e-02,
            7.9689343489e-04)


def _fc_poly(u):
    acc = np.float32(_FC_COEF[6])
    for c in _FC_COEF[5::-1]:
        acc = acc * u + np.float32(c)
    return jnp.where(u <= 1.0, acc, 0.0)


def _aev_body(species_ref, coords_ref, coords_c_ref, posj_ref, posk_ref,
              pidx_ref, dgj_ref, dgk_ref, out_ref):
    ci = pl.program_id(1)
    i0 = ci * _CI
    pi = np.float32(np.pi)

    sp = species_ref[0, 0, :]
    pos = coords_ref[0, :, :]
    pos_c = coords_c_ref[0, 0, :, :]

    diff_c = pos_c[:, :, None] - pos[:, None, :]
    d2_c = jnp.sum(diff_c * diff_c, axis=0)
    crow = jax.lax.broadcasted_iota(jnp.int32, (_CI, _A), 0) + i0
    ccol = jax.lax.broadcasted_iota(jnp.int32, (_CI, _A), 1)
    diag_c = (crow == ccol)
    notdiag_c = (~diag_c).astype(jnp.float32)
    d_c = jnp.sqrt(d2_c + diag_c.astype(jnp.float32))

    fc_r = _fc_poly(d2_c * np.float32(1.0 / (_RCR * _RCR)))
    fc_r = fc_r * notdiag_c
    shfr = (0.9 + 0.26875
            * jax.lax.broadcasted_iota(jnp.int32, (1, 1, 16), 2)
            .astype(jnp.float32))
    rad = 0.25 * jnp.exp(-_ETA_R * (d_c[:, :, None] - shfr) ** 2)
    rad = rad * fc_r[:, :, None]
    sidx = jax.lax.broadcasted_iota(jnp.int32, (_A, _NUM_SPECIES), 1)
    oh = (sp[:, None] == sidx).astype(jnp.float32)
    rad_st = jax.lax.dot_general(oh, rad, (((0,), (1,)), ((), ())),
                                 preferred_element_type=jnp.float32)
    radial = jnp.transpose(rad_st, (1, 0, 2)).reshape(_CI, _RADIAL_F)

    posj = posj_ref[0, :, :]
    posk = posk_ref[0, :, :]
    diag_ij = dgj_ref[0, :, :]
    diag_ik = dgk_ref[0, :, :]

    dj = pos_c[:, :, None] - posj[:, None, :]
    d2_ij = jnp.sum(dj * dj, axis=0)
    dk = pos_c[:, :, None] - posk[:, None, :]
    d2_ik = jnp.sum(dk * dk, axis=0)
    ejk = posj - posk
    e2 = ejk * ejk
    d2_jk = e2[0:1, :] + e2[1:2, :] + e2[2:3, :]

    s2_ij = d2_ij + diag_ij
    s2_ik = d2_ik + diag_ik
    inv_ij = jax.lax.rsqrt(s2_ij)
    inv_ik = jax.lax.rsqrt(s2_ik)
    d_ij = s2_ij * inv_ij
    d_ik = s2_ik * inv_ik

    rca2inv = np.float32(1.0 / (_RCA * _RCA))
    fc_ij = _fc_poly(d2_ij * rca2inv) * (1.0 - diag_ij)
    fc_ik = _fc_poly(d2_ik * rca2inv) * (1.0 - diag_ik)
    w = 2.0 * fc_ij * fc_ik

    dotv = 0.5 * (d2_ij + d2_ik - d2_jk)
    cth = 0.95 * dotv * inv_ij * inv_ik
    sth = jnp.sqrt(jnp.maximum(1.0 - cth * cth, 0.0))
    dsum = jnp.minimum(0.5 * (d_ij + d_ik), 4.0)

    f2_0 = jnp.exp(-_ETA_A * (dsum - 0.9) ** 2)
    r = jnp.exp(10.4 * dsum)
    f2_1 = f2_0 * (r * np.float32(np.exp(-10.4 * 0.9 - 3.38)))
    f2_2 = f2_1 * (r * np.float32(np.exp(-10.4 * 1.55 - 3.38)))
    f2_3 = f2_2 * (r * np.float32(np.exp(-10.4 * 2.2 - 3.38)))
    wf2 = jnp.stack([w * f2_0, w * f2_1, w * f2_2, w * f2_3],
                    axis=1)

    shfz = ((pi / 16.0) + (pi / 8.0)
            * jax.lax.broadcasted_iota(jnp.int32, (1, 8, 1), 1)
            .astype(jnp.float32))
    czh = 0.5 * jnp.cos(shfz)
    szh = 0.5 * jnp.sin(shfz)
    base = 0.5 + cth[:, None, :] * czh + sth[:, None, :] * szh
    f1 = base * base
    f1 = f1 * f1
    f1 = f1 * f1
    f1 = f1 * f1
    f1 = f1 * f1

    ang = wf2[:, :, None, :] * f1[:, None, :, :]
    ang = ang.reshape(_CI, 32, _NQ)

    pidx = jnp.broadcast_to(pidx_ref[0, :, :], (_NUM_PAIRS, _NQ))
    pslot = jax.lax.broadcasted_iota(jnp.int32, (_NUM_PAIRS, _NQ), 0)
    p_oht = (pidx == pslot).astype(jnp.float32)

    ang_p = jax.lax.dot_general(p_oht, ang, (((1,), (2,)), ((), ())),
                                preferred_element_type=jnp.float32)
    angular = jnp.transpose(ang_p, (1, 0, 2)).reshape(_CI, _ANGULAR_F)

    out_ref[0, :, :] = jnp.concatenate([radial, angular], axis=1)


@jax.jit
def _aev_pallas(species, coordinates):
    M, A = species.shape
    sp3 = species.astype(jnp.int32).reshape(M, 1, A)
    coords_t = jnp.transpose(coordinates, (0, 2, 1))
    coords_chunks = jnp.transpose(
        coords_t.reshape(M, 3, A // _CI, _CI), (0, 2, 1, 3))

    jq = jnp.asarray(_JQ, dtype=jnp.int32)
    kq = jnp.asarray(_KQ, dtype=jnp.int32)
    npad = _NQ - _NPAIR
    posj = jnp.concatenate(
        [jnp.take(coords_t, jq, axis=2),
         jnp.full((M, 3, npad), 1.0e4, jnp.float32)], axis=2)
    posk = jnp.concatenate(
        [jnp.take(coords_t, kq, axis=2),
         jnp.full((M, 3, npad), 2.0e4, jnp.float32)], axis=2)

    spi = species.astype(jnp.int32)
    spj = jnp.take(spi, jq, axis=1)
    spk = jnp.take(spi, kq, axis=1)
    mn = jnp.minimum(spj, spk)
    mx = jnp.maximum(spj, spk)
    pidx = (mn * (7 - mn)) // 2 + mx
    pidx = jnp.pad(pidx, ((0, 0), (0, npad))).reshape(M, 1, _NQ)

    jq_pad = np.pad(_JQ, (0, npad), constant_values=-1)
    kq_pad = np.pad(_KQ, (0, npad), constant_values=-1)
    irows = np.arange(A)[:, None]
    dgj = jnp.asarray((jq_pad[None, :] == irows).astype(np.float32)
                      ).reshape(1, A, _NQ)
    dgk = jnp.asarray((kq_pad[None, :] == irows).astype(np.float32)
                      ).reshape(1, A, _NQ)

    out = pl.pallas_call(
        _aev_body,
        grid=(M, A // _CI),
        in_specs=[
            pl.BlockSpec((1, 1, A), lambda m, c: (m, 0, 0)),
            pl.BlockSpec((1, 3, A), lambda m, c: (m, 0, 0)),
            pl.BlockSpec((1, 1, 3, _CI), lambda m, c: (m, c, 0, 0)),
            pl.BlockSpec((1, 3, _NQ), lambda m, c: (m, 0, 0)),
            pl.BlockSpec((1, 3, _NQ), lambda m, c: (m, 0, 0)),
            pl.BlockSpec((1, 1, _NQ), lambda m, c: (m, 0, 0)),
            pl.BlockSpec((1, _CI, _NQ), lambda m, c: (0, c, 0)),
            pl.BlockSpec((1, _CI, _NQ), lambda m, c: (0, c, 0)),
        ],
        out_specs=pl.BlockSpec((1, _CI, _RADIAL_F + _ANGULAR_F),
                               lambda m, c: (m, c, 0)),
        out_shape=jax.ShapeDtypeStruct((M, A, _RADIAL_F + _ANGULAR_F),
                                       jnp.float32),
    )(sp3, coords_t, coords_chunks, posj, posk, pidx, dgj, dgk)
    return out


def kernel(species, coordinates):
    aev = _aev_pallas(species, coordinates)
    return (species, aev)


# speedup vs baseline: 14.7646x; 1.1720x over previous
import functools

import jax
import jax.numpy as jnp
import numpy as np
from jax.experimental import pallas as pl

_RCR = 5.2
_RCA = 3.5
_NUM_SPECIES = 4
_NUM_PAIRS = 10
_ETA_R = 16.0
_ETA_A = 8.0
_A = 32
_NQ = 512
_NPAIR = _A * (_A - 1) // 2
_CI = 32
_RADIAL_F = _NUM_SPECIES * 16
_ANGULAR_F = _NUM_PAIRS * 4 * 8

_JQ, _KQ = np.triu_indices(_A, k=1)

_FC_COEF = (9.9999998695e-01, -2.4674003665e+00, 2.0293461123e+00,
            -6.6757576357e-01, 1.1751096555e-01, -1.2677815461e-02,
            7.9689343489e-04)


def _fc_poly(u):
    acc = np.float32(_FC_COEF[6])
    for c in _FC_COEF[5::-1]:
        acc = acc * u + np.float32(c)
    return jnp.where(u <= 1.0, acc, 0.0)


def _aev_body(species_ref, coords_ref, coords_c_ref, posj_ref, posk_ref,
              pidx_ref, dgj_ref, dgk_ref, posr_ref, shfr_ref, dgr_ref,
              out_ref):
    pi = np.float32(np.pi)

    sp = species_ref[0, 0, :]
    pos = coords_ref[0, :, :]
    pos_c = coords_c_ref[0, 0, :, :]

    posr = posr_ref[0, :, :]
    shfr = shfr_ref[0, :, :]
    dgr = dgr_ref[0, :, :]

    djr = pos[:, :, None] - posr[:, None, :]
    d2_r = jnp.sum(djr * djr, axis=0)
    fc_rf = _fc_poly(d2_r * np.float32(1.0 / (_RCR * _RCR)))
    fc_rf = fc_rf * (0.25 * (1.0 - dgr))
    d_r = jnp.sqrt(d2_r + dgr)
    rad_f = jnp.exp(-_ETA_R * (d_r - shfr) ** 2) * fc_rf
    sidx = jax.lax.broadcasted_iota(jnp.int32, (_A, _NUM_SPECIES), 1)
    oh = (sp[:, None] == sidx).astype(jnp.float32)
    rad_sf = jax.lax.dot_general(oh, rad_f, (((0,), (0,)), ((), ())),
                                 preferred_element_type=jnp.float32)
    radial = jnp.transpose(rad_sf.reshape(_NUM_SPECIES, _CI, 16),
                           (1, 0, 2)).reshape(_CI, _RADIAL_F)

    posj = posj_ref[0, :, :]
    posk = posk_ref[0, :, :]
    diag_ij = dgj_ref[0, :, :]
    diag_ik = dgk_ref[0, :, :]

    dj = pos_c[:, :, None] - posj[:, None, :]
    d2_ij = jnp.sum(dj * dj, axis=0)
    dk = pos_c[:, :, None] - posk[:, None, :]
    d2_ik = jnp.sum(dk * dk, axis=0)
    ejk = posj - posk
    e2 = ejk * ejk
    d2_jk = e2[0:1, :] + e2[1:2, :] + e2[2:3, :]

    s2_ij = d2_ij + diag_ij
    s2_ik = d2_ik + diag_ik
    inv_ij = jax.lax.rsqrt(s2_ij)
    inv_ik = jax.lax.rsqrt(s2_ik)
    d_ij = s2_ij * inv_ij
    d_ik = s2_ik * inv_ik

    rca2inv = np.float32(1.0 / (_RCA * _RCA))
    fc_ij = _fc_poly(d2_ij * rca2inv) * (1.0 - diag_ij)
    fc_ik = _fc_poly(d2_ik * rca2inv) * (1.0 - diag_ik)
    w = 2.0 * fc_ij * fc_ik

    dotv = 0.5 * (d2_ij + d2_ik - d2_jk)
    cth = 0.95 * dotv * inv_ij * inv_ik
    sth = jnp.sqrt(jnp.maximum(1.0 - cth * cth, 0.0))
    dsum = jnp.minimum(0.5 * (d_ij + d_ik), 4.0)

    f2_0 = jnp.exp(-_ETA_A * (dsum - 0.9) ** 2)
    r = jnp.exp(10.4 * dsum)
    f2_1 = f2_0 * (r * np.float32(np.exp(-10.4 * 0.9 - 3.38)))
    f2_2 = f2_1 * (r * np.float32(np.exp(-10.4 * 1.55 - 3.38)))
    f2_3 = f2_2 * (r * np.float32(np.exp(-10.4 * 2.2 - 3.38)))

    shfz = ((pi / 16.0) + (pi / 8.0)
            * jax.lax.broadcasted_iota(jnp.int32, (1, 8, 1), 1)
            .astype(jnp.float32))
    czh = 0.5 * jnp.cos(shfz)
    szh = 0.5 * jnp.sin(shfz)
    base = 0.5 + cth[:, None, :] * czh + sth[:, None, :] * szh
    f1 = base * base
    f1 = f1 * f1
    f1 = f1 * f1
    f1 = f1 * f1
    f1 = f1 * f1

    ang = jnp.concatenate(
        [(w * f2_0)[:, None, :] * f1, (w * f2_1)[:, None, :] * f1,
         (w * f2_2)[:, None, :] * f1, (w * f2_3)[:, None, :] * f1],
        axis=1)

    pidx = jnp.broadcast_to(pidx_ref[0, :, :], (_NUM_PAIRS, _NQ))
    pslot = jax.lax.broadcasted_iota(jnp.int32, (_NUM_PAIRS, _NQ), 0)
    p_oht = (pidx == pslot).astype(jnp.float32)

    ang_p = jax.lax.dot_general(p_oht, ang, (((1,), (2,)), ((), ())),
                                preferred_element_type=jnp.float32)
    angular = jnp.transpose(ang_p, (1, 0, 2)).reshape(_CI, _ANGULAR_F)

    out_ref[0, :, :] = jnp.concatenate([radial, angular], axis=1)


@jax.jit
def _aev_pallas(species, coordinates):
    M, A = species.shape
    sp3 = species.astype(jnp.int32).reshape(M, 1, A)
    coords_t = jnp.transpose(coordinates, (0, 2, 1))
    coords_chunks = jnp.transpose(
        coords_t.reshape(M, 3, A // _CI, _CI), (0, 2, 1, 3))

    jq = jnp.asarray(_JQ, dtype=jnp.int32)
    kq = jnp.asarray(_KQ, dtype=jnp.int32)
    npad = _NQ - _NPAIR
    posj = jnp.concatenate(
        [jnp.take(coords_t, jq, axis=2),
         jnp.full((M, 3, npad), 1.0e4, jnp.float32)], axis=2)
    posk = jnp.concatenate(
        [jnp.take(coords_t, kq, axis=2),
         jnp.full((M, 3, npad), 2.0e4, jnp.float32)], axis=2)

    spi = species.astype(jnp.int32)
    spj = jnp.take(spi, jq, axis=1)
    spk = jnp.take(spi, kq, axis=1)
    mn = jnp.minimum(spj, spk)
    mx = jnp.maximum(spj, spk)
    pidx = (mn * (7 - mn)) // 2 + mx
    pidx = jnp.pad(pidx, ((0, 0), (0, npad))).reshape(M, 1, _NQ)

    jq_pad = np.pad(_JQ, (0, npad), constant_values=-1)
    kq_pad = np.pad(_KQ, (0, npad), constant_values=-1)
    irows = np.arange(A)[:, None]
    dgj = jnp.asarray((jq_pad[None, :] == irows).astype(np.float32)
                      ).reshape(1, A, _NQ)
    dgk = jnp.asarray((kq_pad[None, :] == irows).astype(np.float32)
                      ).reshape(1, A, _NQ)

    posr = jnp.repeat(coords_t, 16, axis=2)
    shfr_np = np.tile(0.9 + 0.26875 * np.arange(16, dtype=np.float32), A)
    shfr_flat = jnp.asarray(shfr_np).reshape(1, 1, A * 16)
    dgr = jnp.asarray(
        ((np.arange(A * 16) // 16)[None, :] == irows).astype(np.float32)
    ).reshape(1, A, A * 16)

    out = pl.pallas_call(
        _aev_body,
        grid=(M, A // _CI),
        in_specs=[
            pl.BlockSpec((1, 1, A), lambda m, c: (m, 0, 0)),
            pl.BlockSpec((1, 3, A), lambda m, c: (m, 0, 0)),
            pl.BlockSpec((1, 1, 3, _CI), lambda m, c: (m, c, 0, 0)),
            pl.BlockSpec((1, 3, _NQ), lambda m, c: (m, 0, 0)),
            pl.BlockSpec((1, 3, _NQ), lambda m, c: (m, 0, 0)),
            pl.BlockSpec((1, 1, _NQ), lambda m, c: (m, 0, 0)),
            pl.BlockSpec((1, _CI, _NQ), lambda m, c: (0, c, 0)),
            pl.BlockSpec((1, _CI, _NQ), lambda m, c: (0, c, 0)),
            pl.BlockSpec((1, 3, A * 16), lambda m, c: (m, 0, 0)),
            pl.BlockSpec((1, 1, A * 16), lambda m, c: (0, 0, 0)),
            pl.BlockSpec((1, A, A * 16), lambda m, c: (0, 0, 0)),
        ],
        out_specs=pl.BlockSpec((1, _CI, _RADIAL_F + _ANGULAR_F),
                               lambda m, c: (m, c, 0)),
        out_shape=jax.ShapeDtypeStruct((M, A, _RADIAL_F + _ANGULAR_F),
                                       jnp.float32),
    )(sp3, coords_t, coords_chunks, posj, posk, pidx, dgj, dgk,
      posr, shfr_flat, dgr)
    return out


def kernel(species, coordinates):
    aev = _aev_pallas(species, coordinates)
    return (species, aev)


# speedup vs baseline: 16.8772x; 1.1431x over previous
import functools

import jax
import jax.numpy as jnp
import numpy as np
from jax.experimental import pallas as pl

_RCR = 5.2
_RCA = 3.5
_NUM_SPECIES = 4
_NUM_PAIRS = 10
_ETA_R = 16.0
_ETA_A = 8.0
_A = 32
_NQ = 512
_NPAIR = _A * (_A - 1) // 2
_NL = _A * 16
_MB = 4
_RADIAL_F = _NUM_SPECIES * 16
_ANGULAR_F = _NUM_PAIRS * 4 * 8

_JQ, _KQ = np.triu_indices(_A, k=1)

_FC_COEF = (9.9999998695e-01, -2.4674003665e+00, 2.0293461123e+00,
            -6.6757576357e-01, 1.1751096555e-01, -1.2677815461e-02,
            7.9689343489e-04)


def _fc_poly(u):
    acc = np.float32(_FC_COEF[6])
    for c in _FC_COEF[5::-1]:
        acc = acc * u + np.float32(c)
    return jnp.where(u <= 1.0, acc, 0.0)


def _aev_body(species_ref, coords_ref, posj_ref, posk_ref, pidx_ref,
              dgj_ref, dgk_ref, posr_ref, shfr_ref, dgr_ref, out_ref):
    pi = np.float32(np.pi)

    sp = species_ref[:, 0, :]
    pos = coords_ref[:, :, :]

    posr = posr_ref[:, :, :]
    shfr = shfr_ref[0, :, :]
    dgr = dgr_ref[0, :, :]

    djr = pos[:, :, :, None] - posr[:, :, None, :]
    d2_r = jnp.sum(djr * djr, axis=1)
    fc_rf = _fc_poly(d2_r * np.float32(1.0 / (_RCR * _RCR)))
    fc_rf = fc_rf * (0.25 * (1.0 - dgr))
    d_r = jnp.sqrt(d2_r + dgr)
    rad_f = jnp.exp(-_ETA_R * (d_r - shfr) ** 2) * fc_rf
    sidx = jax.lax.broadcasted_iota(jnp.int32, (_MB, _A, _NUM_SPECIES), 2)
    oh = (sp[:, :, None] == sidx).astype(jnp.float32)
    rad_sf = jax.lax.dot_general(oh, rad_f, (((1,), (1,)), ((0,), (0,))),
                                 preferred_element_type=jnp.float32)
    radial = jnp.transpose(rad_sf.reshape(_MB, _NUM_SPECIES, _A, 16),
                           (0, 2, 1, 3)).reshape(_MB, _A, _RADIAL_F)

    posj = posj_ref[:, :, :]
    posk = posk_ref[:, :, :]
    diag_ij = dgj_ref[0, :, :][None]
    diag_ik = dgk_ref[0, :, :][None]

    dj = pos[:, :, :, None] - posj[:, :, None, :]
    d2_ij = jnp.sum(dj * dj, axis=1)
    dk = pos[:, :, :, None] - posk[:, :, None, :]
    d2_ik = jnp.sum(dk * dk, axis=1)
    ejk = posj - posk
    e2 = ejk * ejk
    d2_jk = (e2[:, 0:1, :] + e2[:, 1:2, :] + e2[:, 2:3, :])

    s2_ij = d2_ij + diag_ij
    s2_ik = d2_ik + diag_ik
    inv_ij = jax.lax.rsqrt(s2_ij)
    inv_ik = jax.lax.rsqrt(s2_ik)
    d_ij = s2_ij * inv_ij
    d_ik = s2_ik * inv_ik

    rca2inv = np.float32(1.0 / (_RCA * _RCA))
    fc_ij = _fc_poly(d2_ij * rca2inv) * (1.0 - diag_ij)
    fc_ik = _fc_poly(d2_ik * rca2inv) * (1.0 - diag_ik)
    w = 2.0 * fc_ij * fc_ik

    dotv = 0.5 * (d2_ij + d2_ik - d2_jk)
    cth = 0.95 * dotv * inv_ij * inv_ik
    sth = jnp.sqrt(jnp.maximum(1.0 - cth * cth, 0.0))
    dsum = jnp.minimum(0.5 * (d_ij + d_ik), 4.0)

    f2_0 = jnp.exp(-_ETA_A * (dsum - 0.9) ** 2)
    r = jnp.exp(10.4 * dsum)
    f2_1 = f2_0 * (r * np.float32(np.exp(-10.4 * 0.9 - 3.38)))
    f2_2 = f2_1 * (r * np.float32(np.exp(-10.4 * 1.55 - 3.38)))
    f2_3 = f2_2 * (r * np.float32(np.exp(-10.4 * 2.2 - 3.38)))

    shfz = ((pi / 16.0) + (pi / 8.0)
            * jax.lax.broadcasted_iota(jnp.int32, (1, 1, 8, 1), 2)
            .astype(jnp.float32))
    czh = 0.5 * jnp.cos(shfz)
    szh = 0.5 * jnp.sin(shfz)
    base = 0.5 + cth[:, :, None, :] * czh + sth[:, :, None, :] * szh
    f1 = base * base
    f1 = f1 * f1
    f1 = f1 * f1
    f1 = f1 * f1
    f1 = f1 * f1

    ang = jnp.concatenate(
        [(w * f2_0)[:, :, None, :] * f1, (w * f2_1)[:, :, None, :] * f1,
         (w * f2_2)[:, :, None, :] * f1, (w * f2_3)[:, :, None, :] * f1],
        axis=2)
    ang = ang.reshape(_MB, _A * 32, _NQ)

    pidx = jnp.broadcast_to(pidx_ref[:, :, :], (_MB, _NUM_PAIRS, _NQ))
    pslot = jax.lax.broadcasted_iota(jnp.int32, (_MB, _NUM_PAIRS, _NQ), 1)
    p_oht = (pidx == pslot).astype(jnp.float32)

    ang_p = jax.lax.dot_general(p_oht, ang, (((2,), (2,)), ((0,), (0,))),
                                preferred_element_type=jnp.float32)
    angular = jnp.transpose(ang_p.reshape(_MB, _NUM_PAIRS, _A, 32),
                            (0, 2, 1, 3)).reshape(_MB, _A, _ANGULAR_F)

    out_ref[:, :, :] = jnp.concatenate([radial, angular], axis=2)


@jax.jit
def _aev_pallas(species, coordinates):
    M, A = species.shape
    sp3 = species.astype(jnp.int32).reshape(M, 1, A)
    coords_t = jnp.transpose(coordinates, (0, 2, 1))

    jq = jnp.asarray(_JQ, dtype=jnp.int32)
    kq = jnp.asarray(_KQ, dtype=jnp.int32)
    npad = _NQ - _NPAIR
    posj = jnp.concatenate(
        [jnp.take(coords_t, jq, axis=2),
         jnp.full((M, 3, npad), 1.0e4, jnp.float32)], axis=2)
    posk = jnp.concatenate(
        [jnp.take(coords_t, kq, axis=2),
         jnp.full((M, 3, npad), 2.0e4, jnp.float32)], axis=2)

    spi = species.astype(jnp.int32)
    spj = jnp.take(spi, jq, axis=1)
    spk = jnp.take(spi, kq, axis=1)
    mn = jnp.minimum(spj, spk)
    mx = jnp.maximum(spj, spk)
    pidx = (mn * (7 - mn)) // 2 + mx
    pidx = jnp.pad(pidx, ((0, 0), (0, npad))).reshape(M, 1, _NQ)

    jq_pad = np.pad(_JQ, (0, npad), constant_values=-1)
    kq_pad = np.pad(_KQ, (0, npad), constant_values=-1)
    irows = np.arange(A)[:, None]
    dgj = jnp.asarray((jq_pad[None, :] == irows).astype(np.float32)
                      ).reshape(1, A, _NQ)
    dgk = jnp.asarray((kq_pad[None, :] == irows).astype(np.float32)
                      ).reshape(1, A, _NQ)

    posr = jnp.repeat(coords_t, 16, axis=2)
    shfr_np = np.tile(0.9 + 0.26875 * np.arange(16, dtype=np.float32), A)
    shfr_flat = jnp.asarray(shfr_np).reshape(1, 1, _NL)
    dgr = jnp.asarray(
        ((np.arange(_NL) // 16)[None, :] == irows).astype(np.float32)
    ).reshape(1, A, _NL)

    out = pl.pallas_call(
        _aev_body,
        grid=(M // _MB,),
        in_specs=[
            pl.BlockSpec((_MB, 1, A), lambda m: (m, 0, 0)),
            pl.BlockSpec((_MB, 3, A), lambda m: (m, 0, 0)),
            pl.BlockSpec((_MB, 3, _NQ), lambda m: (m, 0, 0)),
            pl.BlockSpec((_MB, 3, _NQ), lambda m: (m, 0, 0)),
            pl.BlockSpec((_MB, 1, _NQ), lambda m: (m, 0, 0)),
            pl.BlockSpec((1, A, _NQ), lambda m: (0, 0, 0)),
            pl.BlockSpec((1, A, _NQ), lambda m: (0, 0, 0)),
            pl.BlockSpec((_MB, 3, _NL), lambda m: (m, 0, 0)),
            pl.BlockSpec((1, 1, _NL), lambda m: (0, 0, 0)),
            pl.BlockSpec((1, A, _NL), lambda m: (0, 0, 0)),
        ],
        out_specs=pl.BlockSpec((_MB, A, _RADIAL_F + _ANGULAR_F),
                               lambda m: (m, 0, 0)),
        out_shape=jax.ShapeDtypeStruct((M, A, _RADIAL_F + _ANGULAR_F),
                                       jnp.float32),
    )(sp3, coords_t, posj, posk, pidx, dgj, dgk, posr, shfr_flat, dgr)
    return out


def kernel(species, coordinates):
    aev = _aev_pallas(species, coordinates)
    return (species, aev)


# speedup vs baseline: 17.1690x; 1.0173x over previous
import functools

import jax
import jax.numpy as jnp
import numpy as np
from jax.experimental import pallas as pl

_RCR = 5.2
_RCA = 3.5
_NUM_SPECIES = 4
_NUM_PAIRS = 10
_ETA_R = 16.0
_ETA_A = 8.0
_A = 32
_NQ = 512
_NPAIR = _A * (_A - 1) // 2
_NL = _A * 16
_MB = 8
_RADIAL_F = _NUM_SPECIES * 16
_ANGULAR_F = _NUM_PAIRS * 4 * 8

_JQ, _KQ = np.triu_indices(_A, k=1)

_FC_COEF = (9.9999998695e-01, -2.4674003665e+00, 2.0293461123e+00,
            -6.6757576357e-01, 1.1751096555e-01, -1.2677815461e-02,
            7.9689343489e-04)


def _fc_poly(u):
    acc = np.float32(_FC_COEF[6])
    for c in _FC_COEF[5::-1]:
        acc = acc * u + np.float32(c)
    return jnp.where(u <= 1.0, acc, 0.0)


def _aev_body(species_ref, coords_ref, posj_ref, posk_ref, pidx_ref,
              dgj_ref, dgk_ref, posr_ref, shfr_ref, dgr_ref, out_ref):
    pi = np.float32(np.pi)

    sp = species_ref[:, 0, :]
    pos = coords_ref[:, :, :]

    posr = posr_ref[:, :, :]
    shfr = shfr_ref[0, :, :]
    dgr = dgr_ref[0, :, :]

    djr = pos[:, :, :, None] - posr[:, :, None, :]
    d2_r = jnp.sum(djr * djr, axis=1)
    fc_rf = _fc_poly(d2_r * np.float32(1.0 / (_RCR * _RCR)))
    fc_rf = fc_rf * (0.25 * (1.0 - dgr))
    d_r = jnp.sqrt(d2_r + dgr)
    rad_f = jnp.exp(-_ETA_R * (d_r - shfr) ** 2) * fc_rf
    sidx = jax.lax.broadcasted_iota(jnp.int32, (_MB, _A, _NUM_SPECIES), 2)
    oh = (sp[:, :, None] == sidx).astype(jnp.float32)
    rad_sf = jax.lax.dot_general(oh, rad_f, (((1,), (1,)), ((0,), (0,))),
                                 preferred_element_type=jnp.float32)
    radial = jnp.transpose(rad_sf.reshape(_MB, _NUM_SPECIES, _A, 16),
                           (0, 2, 1, 3)).reshape(_MB, _A, _RADIAL_F)

    posj = posj_ref[:, :, :]
    posk = posk_ref[:, :, :]
    diag_ij = dgj_ref[0, :, :][None]
    diag_ik = dgk_ref[0, :, :][None]

    dj = pos[:, :, :, None] - posj[:, :, None, :]
    d2_ij = jnp.sum(dj * dj, axis=1)
    dk = pos[:, :, :, None] - posk[:, :, None, :]
    d2_ik = jnp.sum(dk * dk, axis=1)
    ejk = posj - posk
    e2 = ejk * ejk
    d2_jk = (e2[:, 0:1, :] + e2[:, 1:2, :] + e2[:, 2:3, :])

    s2_ij = d2_ij + diag_ij
    s2_ik = d2_ik + diag_ik
    inv_ij = jax.lax.rsqrt(s2_ij)
    inv_ik = jax.lax.rsqrt(s2_ik)
    d_ij = s2_ij * inv_ij
    d_ik = s2_ik * inv_ik

    rca2inv = np.float32(1.0 / (_RCA * _RCA))
    fc_ij = _fc_poly(d2_ij * rca2inv) * (1.0 - diag_ij)
    fc_ik = _fc_poly(d2_ik * rca2inv) * (1.0 - diag_ik)
    w = 2.0 * fc_ij * fc_ik

    dotv = 0.5 * (d2_ij + d2_ik - d2_jk)
    cth = 0.95 * dotv * inv_ij * inv_ik
    sth = jnp.sqrt(jnp.maximum(1.0 - cth * cth, 0.0))
    dsum = jnp.minimum(0.5 * (d_ij + d_ik), 4.0)

    f2_0 = jnp.exp(-_ETA_A * (dsum - 0.9) ** 2)
    r = jnp.exp(10.4 * dsum)
    f2_1 = f2_0 * (r * np.float32(np.exp(-10.4 * 0.9 - 3.38)))
    f2_2 = f2_1 * (r * np.float32(np.exp(-10.4 * 1.55 - 3.38)))
    f2_3 = f2_2 * (r * np.float32(np.exp(-10.4 * 2.2 - 3.38)))

    shfz = ((pi / 16.0) + (pi / 8.0)
            * jax.lax.broadcasted_iota(jnp.int32, (1, 1, 8, 1), 2)
            .astype(jnp.float32))
    czh = 0.5 * jnp.cos(shfz)
    szh = 0.5 * jnp.sin(shfz)
    base = 0.5 + cth[:, :, None, :] * czh + sth[:, :, None, :] * szh
    f1 = base * base
    f1 = f1 * f1
    f1 = f1 * f1
    f1 = f1 * f1
    f1 = f1 * f1

    ang = jnp.concatenate(
        [(w * f2_0)[:, :, None, :] * f1, (w * f2_1)[:, :, None, :] * f1,
         (w * f2_2)[:, :, None, :] * f1, (w * f2_3)[:, :, None, :] * f1],
        axis=2)
    ang = ang.reshape(_MB, _A * 32, _NQ)

    pidx = jnp.broadcast_to(pidx_ref[:, :, :], (_MB, _NUM_PAIRS, _NQ))
    pslot = jax.lax.broadcasted_iota(jnp.int32, (_MB, _NUM_PAIRS, _NQ), 1)
    p_oht = (pidx == pslot).astype(jnp.float32)

    ang_p = jax.lax.dot_general(p_oht, ang, (((2,), (2,)), ((0,), (0,))),
                                preferred_element_type=jnp.float32)
    angular = jnp.transpose(ang_p.reshape(_MB, _NUM_PAIRS, _A, 32),
                            (0, 2, 1, 3)).reshape(_MB, _A, _ANGULAR_F)

    out_ref[:, :, :] = jnp.concatenate([radial, angular], axis=2)


@jax.jit
def _aev_pallas(species, coordinates):
    M, A = species.shape
    sp3 = species.astype(jnp.int32).reshape(M, 1, A)
    coords_t = jnp.transpose(coordinates, (0, 2, 1))

    jq = jnp.asarray(_JQ, dtype=jnp.int32)
    kq = jnp.asarray(_KQ, dtype=jnp.int32)
    npad = _NQ - _NPAIR
    posj = jnp.concatenate(
        [jnp.take(coords_t, jq, axis=2),
         jnp.full((M, 3, npad), 1.0e4, jnp.float32)], axis=2)
    posk = jnp.concatenate(
        [jnp.take(coords_t, kq, axis=2),
         jnp.full((M, 3, npad), 2.0e4, jnp.float32)], axis=2)

    spi = species.astype(jnp.int32)
    spj = jnp.take(spi, jq, axis=1)
    spk = jnp.take(spi, kq, axis=1)
    mn = jnp.minimum(spj, spk)
    mx = jnp.maximum(spj, spk)
    pidx = (mn * (7 - mn)) // 2 + mx
    pidx = jnp.pad(pidx, ((0, 0), (0, npad))).reshape(M, 1, _NQ)

    jq_pad = np.pad(_JQ, (0, npad), constant_values=-1)
    kq_pad = np.pad(_KQ, (0, npad), constant_values=-1)
    irows = np.arange(A)[:, None]
    dgj = jnp.asarray((jq_pad[None, :] == irows).astype(np.float32)
                      ).reshape(1, A, _NQ)
    dgk = jnp.asarray((kq_pad[None, :] == irows).astype(np.float32)
                      ).reshape(1, A, _NQ)

    posr = jnp.repeat(coords_t, 16, axis=2)
    shfr_np = np.tile(0.9 + 0.26875 * np.arange(16, dtype=np.float32), A)
    shfr_flat = jnp.asarray(shfr_np).reshape(1, 1, _NL)
    dgr = jnp.asarray(
        ((np.arange(_NL) // 16)[None, :] == irows).astype(np.float32)
    ).reshape(1, A, _NL)

    out = pl.pallas_call(
        _aev_body,
        grid=(M // _MB,),
        in_specs=[
            pl.BlockSpec((_MB, 1, A), lambda m: (m, 0, 0)),
            pl.BlockSpec((_MB, 3, A), lambda m: (m, 0, 0)),
            pl.BlockSpec((_MB, 3, _NQ), lambda m: (m, 0, 0)),
            pl.BlockSpec((_MB, 3, _NQ), lambda m: (m, 0, 0)),
            pl.BlockSpec((_MB, 1, _NQ), lambda m: (m, 0, 0)),
            pl.BlockSpec((1, A, _NQ), lambda m: (0, 0, 0)),
            pl.BlockSpec((1, A, _NQ), lambda m: (0, 0, 0)),
            pl.BlockSpec((_MB, 3, _NL), lambda m: (m, 0, 0)),
            pl.BlockSpec((1, 1, _NL), lambda m: (0, 0, 0)),
            pl.BlockSpec((1, A, _NL), lambda m: (0, 0, 0)),
        ],
        out_specs=pl.BlockSpec((_MB, A, _RADIAL_F + _ANGULAR_F),
                               lambda m: (m, 0, 0)),
        out_shape=jax.ShapeDtypeStruct((M, A, _RADIAL_F + _ANGULAR_F),
                                       jnp.float32),
    )(sp3, coords_t, posj, posk, pidx, dgj, dgk, posr, shfr_flat, dgr)
    return out


def kernel(species, coordinates):
    aev = _aev_pallas(species, coordinates)
    return (species, aev)
